# baseline (speedup 1.0000x reference)
import functools

import jax
import jax.numpy as jnp
import numpy as np
from jax import lax
from jax.experimental import pallas as pl
from jax.experimental.pallas import tpu as pltpu
from jax.experimental.pallas import tpu_sc as plsc

_N_ENT = 100000
_DIM = 64
_HD = _DIM // 2
_N_EMB = 8
_B = 16384
_B2 = 2 * _B

_NC = 2
_NS = 16
_L = 16
_NW = _NC * _NS
_CPW = _B2 // _NW
_NT = 32
_NSTEP = _CPW // _NT
_SLOTS = 4

_PSI = 1.2
_MARGIN = 1.0
_LAMBDA_POS = 2.0
_LAMBDA_NEG = 2.0

_REPACK_E = 1024


def _pack_rows(x_ref, o_ref, eye, dn):
  for s in range(_SLOTS):
    a = lax.dot_general(x_ref[s], eye, dn,
                        preferred_element_type=jnp.float32)
    u = lax.bitcast_convert_type(a, jnp.int32)
    lo = lax.shift_right_logical(u[:, 0:_HD] + 0x8000, 16)
    hi = (u[:, _HD:_DIM] + 0x8000) & np.int32(-65536)
    o_ref[:, s * _HD:(s + 1) * _HD] = lax.bitcast_convert_type(
        lo | hi, jnp.float32)


def _repack_body(xe_ref, xr_ref, eye_ref, oe_ref, or_ref):
  eye = eye_ref[...]
  dn = (((0,), (0,)), ((), ()))
  _pack_rows(xe_ref, oe_ref, eye, dn)
  _pack_rows(xr_ref, or_ref, eye, dn)


def _repack(ent_t, rel_t, chunk):
  n = (_N_ENT + _REPACK_E - 1) // _REPACK_E
  eye = jnp.asarray(np.eye(_DIM, dtype=np.float32))
  return pl.pallas_call(
      _repack_body,
      grid=(n,),
      in_specs=[
          pl.BlockSpec((_SLOTS, _DIM, _REPACK_E),
                       lambda i, c=chunk: (c, 0, i)),
          pl.BlockSpec((_SLOTS, _DIM, _REPACK_E),
                       lambda i, c=chunk: (c, 0, i)),
          pl.BlockSpec((_DIM, _DIM), lambda i: (0, 0)),
      ],
      out_specs=[
          pl.BlockSpec((_REPACK_E, 2 * _DIM), lambda i: (i, 0)),
          pl.BlockSpec((_REPACK_E, 2 * _DIM), lambda i: (i, 0)),
      ],
      out_shape=[
          jax.ShapeDtypeStruct((_N_ENT, 2 * _DIM), jnp.float32),
          jax.ShapeDtypeStruct((_N_ENT, 2 * _DIM), jnp.float32),
      ],
      compiler_params=pltpu.CompilerParams(
          fuse_transposed_lhs_in_matmul=True),
  )(ent_t, rel_t, eye)


_MASK_HI = np.int32(-65536)


def _sc_body(ent_hbm, rel_hbm, hidx_hbm, tidx_hbm, ridx_hbm, out_hbm,
             raw_h, raw_t, raw_r,
             ix_h_a, ix_t_a, ix_r_a, ix_h_b, ix_t_b, ix_r_b,
             bh_a, bt_a, br_a, bh_b, bt_b, br_b,
             stage, sem_a, sem_b):
  wid = lax.axis_index("s") * _NC + lax.axis_index("c")
  wbase = wid * _CPW

  pltpu.sync_copy(hidx_hbm.at[pl.ds(wbase, _CPW)], raw_h)
  pltpu.sync_copy(tidx_hbm.at[pl.ds(wbase, _CPW)], raw_t)
  pltpu.sync_copy(ridx_hbm.at[pl.ds(wbase, _CPW)], raw_r)

  iota = lax.broadcasted_iota(jnp.int32, (_L,), 0)

  def build_and_fire(s, ix_h, ix_t, ix_r, bh, bt, br, sem):
    base = s * _NT
    for c in range(_NT // _L):
      sl = pl.ds(c * _L, _L)
      bsl = pl.ds(base + c * _L, _L)
      ix_h[sl] = raw_h[bsl]
      ix_t[sl] = raw_t[bsl]
      ix_r[sl] = raw_r[bsl]
    pltpu.async_copy(ent_hbm.at[ix_h], bh, sem)
    pltpu.async_copy(ent_hbm.at[ix_t], bt, sem)
    pltpu.async_copy(rel_hbm.at[ix_r], br, sem)

  def unpack(v):
    u = plsc.bitcast(v, jnp.int32)
    dlo = plsc.bitcast(lax.shift_left(u, 16), jnp.float32)
    dhi = plsc.bitcast(u & _MASK_HI, jnp.float32)
    return dlo, dhi

  def compute(s, ix_h, ix_t, ix_r, bh, bt, br, sem):
    pltpu.make_async_copy(ent_hbm.at[ix_h], bh, sem).wait()
    pltpu.make_async_copy(ent_hbm.at[ix_t], bt, sem).wait()
    pltpu.make_async_copy(rel_hbm.at[ix_r], br, sem).wait()

    def slot_body(k, _):
      cb = k * _HD
      qcol = (s % 4) * _NT

      def group_loop(comb):
        def group_body(g, _):
          rows = g * _L + iota
          acc = jnp.zeros((_L,), jnp.float32)
          for w in range(_HD):
            col = cb + ((iota + w) & (_HD - 1))
            h0, h1 = unpack(plsc.load_gather(bh, [rows, col]))
            t0, t1 = unpack(plsc.load_gather(bt, [rows, col]))
            r0, r1 = unpack(plsc.load_gather(br, [rows, col]))
            v0 = comb(h0, t0, r0)
            v1 = comb(h1, t1, r1)
            acc = acc + v0 * v0 + v1 * v1
          stage[k, pl.ds(qcol + g * _L, _L)] = acc
          return 0
        lax.fori_loop(0, _NT // _L, group_body, 0, unroll=1)

      sh = jnp.where(k == 2, -1.0, 1.0)
      st = jnp.where(k == 0, -1.0, 1.0)
      sr = jnp.where(k == 1, -1.0, 1.0)

      @pl.when(k < 3)
      def _():
        group_loop(lambda h, t, r: sh * h + st * t + sr * r)

      @pl.when(k == 3)
      def _():
        group_loop(lambda h, t, r: h - r * t)

      return 0

    lax.fori_loop(0, _SLOTS, slot_body, 0, unroll=1)

    @pl.when(s % 4 == 3)
    def _():
      pltpu.sync_copy(
          stage, out_hbm.at[:, pl.ds(wbase + (s // 4) * 4 * _NT, 4 * _NT)])

  build_and_fire(0, ix_h_a, ix_t_a, ix_r_a, bh_a, bt_a, br_a, sem_a)

  def macro_body(m, _):
    s_a = 2 * m
    s_b = 2 * m + 1
    build_and_fire(s_b, ix_h_b, ix_t_b, ix_r_b, bh_b, bt_b, br_b, sem_b)
    compute(s_a, ix_h_a, ix_t_a, ix_r_a, bh_a, bt_a, br_a, sem_a)

    @pl.when(m < _NSTEP // 2 - 1)
    def _():
      build_and_fire(s_a + 2, ix_h_a, ix_t_a, ix_r_a, bh_a, bt_a, br_a,
                     sem_a)

    compute(s_b, ix_h_b, ix_t_b, ix_r_b, bh_b, bt_b, br_b, sem_b)
    return 0

  lax.fori_loop(0, _NSTEP // 2, macro_body, 0, unroll=1)


_sc_ssq = functools.partial(
    pl.kernel,
    out_type=jax.ShapeDtypeStruct((_SLOTS, _B2), jnp.float32),
    name="mde_ssq_gather",
    mesh=plsc.VectorSubcoreMesh(
        core_axis_name="c", subcore_axis_name="s",
        num_cores=_NC, num_subcores=_NS),
    compiler_params=pltpu.CompilerParams(
        needs_layout_passes=False, use_tc_tiling_on_sc=True),
    scratch_types=[
        pltpu.VMEM((_CPW,), jnp.int32),
        pltpu.VMEM((_CPW,), jnp.int32),
        pltpu.VMEM((_CPW,), jnp.int32),
        pltpu.VMEM((_NT,), jnp.int32),
        pltpu.VMEM((_NT,), jnp.int32),
        pltpu.VMEM((_NT,), jnp.int32),
        pltpu.VMEM((_NT,), jnp.int32),
        pltpu.VMEM((_NT,), jnp.int32),
        pltpu.VMEM((_NT,), jnp.int32),
        pltpu.VMEM((_NT, 2 * _DIM), jnp.float32),
        pltpu.VMEM((_NT, 2 * _DIM), jnp.float32),
        pltpu.VMEM((_NT, 2 * _DIM), jnp.float32),
        pltpu.VMEM((_NT, 2 * _DIM), jnp.float32),
        pltpu.VMEM((_NT, 2 * _DIM), jnp.float32),
        pltpu.VMEM((_NT, 2 * _DIM), jnp.float32),
        pltpu.VMEM((_SLOTS, 4 * _NT), jnp.float32),
        pltpu.SemaphoreType.DMA,
        pltpu.SemaphoreType.DMA,
    ],
)(_sc_body)


_W = (1.5 / 18.0, 3.0 / 18.0, 1.5 / 18.0, 3.0 / 18.0)


def _tc_body(s0_ref, s1_ref, loss_ref, pos_ref, neg_ref):
  n0 = jnp.sqrt(s0_ref[...])
  n1 = jnp.sqrt(s1_ref[...])
  score = _W[0] * (n0[0:1, :] + n1[0:1, :])
  for k in range(1, _SLOTS):
    score = score + _W[k] * (n0[k:k + 1, :] + n1[k:k + 1, :])
  score = score - _PSI
  pos = jnp.sum(jnp.maximum(score[:, :_B] - (_LAMBDA_POS - _MARGIN), 0.0))
  neg = jnp.sum(jnp.maximum((_LAMBDA_NEG + _MARGIN) - score[:, _B:], 0.0))
  loss_ref[...] = jnp.full((1, 1), pos + neg, jnp.float32)
  pos_ref[...] = jnp.full((1, 1), pos, jnp.float32)
  neg_ref[...] = jnp.full((1, 1), neg, jnp.float32)


def _tc_finish(ssq0, ssq1):
  return pl.pallas_call(
      _tc_body,
      out_shape=(
          jax.ShapeDtypeStruct((1, 1), jnp.float32),
          jax.ShapeDtypeStruct((1, 1), jnp.float32),
          jax.ShapeDtypeStruct((1, 1), jnp.float32),
      ),
  )(ssq0, ssq1)


def kernel(x_train, x_train_negative, entity_emb, relation_emb):
  ent_t = jnp.transpose(entity_emb, (0, 2, 1))
  rel_t = jnp.transpose(relation_emb, (0, 2, 1))
  hidx = jnp.concatenate([x_train[:, 0], x_train_negative[:, 0]]).astype(jnp.int32)
  tidx = jnp.concatenate([x_train[:, 1], x_train_negative[:, 1]]).astype(jnp.int32)
  ridx = jnp.concatenate([x_train[:, 2], x_train_negative[:, 2]]).astype(jnp.int32)
  pe0, pr0 = _repack(ent_t, rel_t, 0)
  ssq0 = _sc_ssq(pe0, pr0, hidx, tidx, ridx)
  pe1, pr1 = _repack(ent_t, rel_t, 1)
  ssq1 = _sc_ssq(pe1, pr1, hidx, tidx, ridx)
  loss, pos, neg = _tc_finish(ssq0, ssq1)
  return (loss[0, 0], pos[0, 0], neg[0, 0])
